# Initial kernel scaffold; baseline (speedup 1.0000x reference)
import jax
import jax.numpy as jnp
from jax.experimental import pallas as pl


def kernel(x_cat, tables, W1, b1, W2, b2, W3, b3):
    raise NotImplementedError("write your pallas kernel here")


# speedup vs baseline: 8.1467x; 8.1467x over previous
import functools

import jax
import jax.numpy as jnp
from jax import lax
from jax.experimental import pallas as pl
from jax.experimental.pallas import tpu as pltpu
from jax.experimental.pallas import tpu_sc as plsc

_B = 16384
_F = 26
_V = 100000
_D = 32
_TOT = _F * _D
_R = _B * _F
_NC, _NS = 2, 16
_NW = _NC * _NS
_ROWS_W = _R // _NW
_CHUNK = 128
_CHUNKS = _ROWS_W // _CHUNK
_NBUF = 8
_GROUPS = _CHUNKS // _NBUF


def _gather_body(idx_hbm, tbl_hbm, out_hbm, idx_v, rows_v, gsem):
    wid = lax.axis_index("s") * _NC + lax.axis_index("c")
    pltpu.sync_copy(idx_hbm.at[wid], idx_v)
    base = wid * _ROWS_W

    def fire(j, b):
        pltpu.async_copy(tbl_hbm.at[idx_v.at[j]], rows_v.at[b], gsem)

    def drain_store(j, b):
        pltpu.make_async_copy(tbl_hbm.at[idx_v.at[0]], rows_v.at[b], gsem).wait()
        pltpu.sync_copy(rows_v.at[b],
                        out_hbm.at[pl.ds(base + j * _CHUNK, _CHUNK)])

    for b in range(_NBUF):
        fire(b, b)

    def group(g, carry):
        for b in range(_NBUF):
            j = g * _NBUF + b
            drain_store(j, b)
            fire(j + _NBUF, b)
        return carry

    lax.fori_loop(0, _GROUPS - 1, group, 0)
    for b in range(_NBUF):
        drain_store((_GROUPS - 1) * _NBUF + b, b)


def _sc_gather(idx_grouped, flat_tbl):
    mesh = plsc.VectorSubcoreMesh(core_axis_name="c", subcore_axis_name="s")
    return pl.kernel(
        _gather_body,
        mesh=mesh,
        compiler_params=pltpu.CompilerParams(use_tc_tiling_on_sc=False),
        out_type=jax.ShapeDtypeStruct((_R, _D), jnp.float32),
        scratch_types=[
            pltpu.VMEM((_CHUNKS, _CHUNK), jnp.int32),
            pltpu.VMEM((_NBUF, _CHUNK, _D), jnp.float32),
            pltpu.SemaphoreType.DMA,
        ],
    )(idx_grouped, flat_tbl)


_BLK = 2048


def _mlp_body(x_ref, w1_ref, b1_ref, w2_ref, b2_ref, w3_ref, b3_ref, o_ref):
    x = x_ref[...]
    h = jnp.maximum(
        jnp.dot(x, w1_ref[...], preferred_element_type=jnp.float32)
        + b1_ref[...], 0.0)
    h = jnp.maximum(
        jnp.dot(h, w2_ref[...], preferred_element_type=jnp.float32)
        + b2_ref[...], 0.0)
    o_ref[...] = (jnp.dot(h, w3_ref[...], preferred_element_type=jnp.float32)
                  + b3_ref[...])


def _mlp(x, W1, b1, W2, b2, W3, b3):
    return pl.pallas_call(
        _mlp_body,
        grid=(_B // _BLK,),
        in_specs=[
            pl.BlockSpec((_BLK, _TOT), lambda i: (i, 0)),
            pl.BlockSpec((_TOT, 32), lambda i: (0, 0)),
            pl.BlockSpec((1, 32), lambda i: (0, 0)),
            pl.BlockSpec((32, 16), lambda i: (0, 0)),
            pl.BlockSpec((1, 16), lambda i: (0, 0)),
            pl.BlockSpec((16, 10), lambda i: (0, 0)),
            pl.BlockSpec((1, 10), lambda i: (0, 0)),
        ],
        out_specs=pl.BlockSpec((_BLK, 10), lambda i: (i, 0)),
        out_shape=jax.ShapeDtypeStruct((_B, 10), jnp.float32),
    )(x, W1, b1.reshape(1, 32), W2, b2.reshape(1, 16),
      W3, b3.reshape(1, 10))


def kernel(x_cat, tables, W1, b1, W2, b2, W3, b3):
    offs = (jnp.arange(_F, dtype=jnp.int32) * _V)[None, :]
    idx = (x_cat.astype(jnp.int32) + offs).reshape(_NW, _CHUNKS, _CHUNK)
    flat_tbl = tables.reshape(_F * _V, _D)
    rows = _sc_gather(idx, flat_tbl)
    x = rows.reshape(_B, _TOT)
    return _mlp(x, W1, b1, W2, b2, W3, b3)


# speedup vs baseline: 10.1979x; 1.2518x over previous
import functools

import jax
import jax.numpy as jnp
from jax import lax
from jax.experimental import pallas as pl
from jax.experimental.pallas import tpu as pltpu
from jax.experimental.pallas import tpu_sc as plsc

_B = 16384
_F = 26
_V = 100000
_D = 32
_TOT = _F * _D
_R = _B * _F
_NC, _NS = 2, 16
_NW = _NC * _NS

_VB = 4096


def _tp_body(x_ref, o_ref):
    x = x_ref[...][0]
    xt = jnp.transpose(x, (1, 0))
    o_ref[...] = jnp.concatenate(
        [xt, jnp.zeros((_VB, 96), jnp.float32)], axis=1)[None]


def _transpose_tbl(tabT):
    return pl.pallas_call(
        _tp_body,
        grid=(_F, (_V + _VB - 1) // _VB),
        in_specs=[pl.BlockSpec((1, _D, _VB), lambda f, v: (f, 0, v))],
        out_specs=pl.BlockSpec((1, _VB, 128), lambda f, v: (f, v, 0)),
        out_shape=jax.ShapeDtypeStruct((_F, _V, 128), jnp.float32),
    )(tabT)


_ROWS_W = _R // _NW
_CHUNK = 128
_NCH = _ROWS_W // _CHUNK
_NBUF = 4


def _gather_body(ridx, tbl, out, rawall, staged, outv, gsem):
    w = lax.axis_index("s") * _NC + lax.axis_index("c")
    base = w * _ROWS_W
    pltpu.sync_copy(ridx.at[pl.ds(base, _ROWS_W)], rawall)

    def extract(b):
        def row(i, carry):
            co = (i & 3) * 32
            outv[i >> 2, pl.ds(co, 16)] = staged[b, i, pl.ds(0, 16)]
            outv[i >> 2, pl.ds(co + 16, 16)] = staged[b, i, pl.ds(16, 16)]
            return carry
        lax.fori_loop(0, _CHUNK, row, 0)

    def fire(j, b):
        pltpu.async_copy(tbl.at[rawall.at[pl.ds(j * _CHUNK, _CHUNK)]],
                         staged.at[b], gsem)

    def proc(j, b):
        pltpu.make_async_copy(tbl.at[rawall.at[pl.ds(0, _CHUNK)]],
                              staged.at[b], gsem).wait()
        extract(b)
        pltpu.sync_copy(outv, out.at[pl.ds(w * 3328 + j * 32, 32)])

    for b in range(_NBUF):
        fire(b, b)

    def grpf(g, carry):
        for b in range(_NBUF):
            j = g * _NBUF + b
            proc(j, b)
            fire(j + _NBUF, b)
        return carry

    lax.fori_loop(0, _NCH // _NBUF - 1, grpf, 0)
    for b in range(_NBUF):
        proc(_NCH - _NBUF + b, b)


def _gather(ridx, tbl128):
    mesh = plsc.VectorSubcoreMesh(core_axis_name="c", subcore_axis_name="s")
    return pl.kernel(
        _gather_body,
        mesh=mesh,
        out_type=jax.ShapeDtypeStruct((_R * _D // 128, 128), jnp.float32),
        scratch_types=[
            pltpu.VMEM((_ROWS_W,), jnp.int32),
            pltpu.VMEM((_NBUF, _CHUNK, 128), jnp.float32),
            pltpu.VMEM((32, 128), jnp.float32),
            pltpu.SemaphoreType.DMA,
        ],
    )(ridx, tbl128)


_BLK = 2048


def _mlp_body(x_ref, w1_ref, b1_ref, w2_ref, b2_ref, w3_ref, b3_ref, o_ref):
    x = x_ref[...]
    h = jnp.maximum(
        jnp.dot(x, w1_ref[...], preferred_element_type=jnp.float32)
        + b1_ref[...], 0.0)
    h = jnp.maximum(
        jnp.dot(h, w2_ref[...], preferred_element_type=jnp.float32)
        + b2_ref[...], 0.0)
    o_ref[...] = (jnp.dot(h, w3_ref[...], preferred_element_type=jnp.float32)
                  + b3_ref[...])


def _mlp(x, W1, b1, W2, b2, W3, b3):
    return pl.pallas_call(
        _mlp_body,
        grid=(_B // _BLK,),
        in_specs=[
            pl.BlockSpec((_BLK, _TOT), lambda i: (i, 0)),
            pl.BlockSpec((_TOT, 32), lambda i: (0, 0)),
            pl.BlockSpec((1, 32), lambda i: (0, 0)),
            pl.BlockSpec((32, 16), lambda i: (0, 0)),
            pl.BlockSpec((1, 16), lambda i: (0, 0)),
            pl.BlockSpec((16, 10), lambda i: (0, 0)),
            pl.BlockSpec((1, 10), lambda i: (0, 0)),
        ],
        out_specs=pl.BlockSpec((_BLK, 10), lambda i: (i, 0)),
        out_shape=jax.ShapeDtypeStruct((_B, 10), jnp.float32),
    )(x, W1, b1.reshape(1, 32), W2, b2.reshape(1, 16),
      W3, b3.reshape(1, 10))


def kernel(x_cat, tables, W1, b1, W2, b2, W3, b3):
    tabT = jnp.transpose(tables, (0, 2, 1))
    tbl128 = _transpose_tbl(tabT).reshape(_F * _V, 128)

    offs = (jnp.arange(_F, dtype=jnp.int32) * _V)[None, :]
    ridx = (x_cat.astype(jnp.int32) + offs).reshape(_R)
    x128 = _gather(ridx, tbl128)
    x = x128.reshape(_B, _TOT)
    return _mlp(x, W1, b1, W2, b2, W3, b3)
